# baseline (speedup 1.0000x reference)
import jax
import jax.numpy as jnp
from jax import lax
from jax.experimental import pallas as pl
from jax.experimental.pallas import tpu as pltpu
from jax.experimental.pallas import tpu_sc as plsc

N = 10000
E = 320000
D = 128
D_OUT = 64

NC = 2
NS = 16
L = 16
NW = NC * NS

C = 125
W = 8
NWIN = 10
CHUNKS = W * NWIN
N_PAD = 10240
RPS = N_PAD // NS
R_BLK = 2000
SUB = 64


def _mesh():
    return plsc.VectorSubcoreMesh(core_axis_name="c", subcore_axis_name="s")


def _deg_body(edge_hbm, out_hbm, ibuf, ones_v, zeros_v, acc, semi, sems):
    c = lax.axis_index("c")
    s = lax.axis_index("s")
    wid = s * NC + c

    def idx_start(w):
        pltpu.make_async_copy(edge_hbm.at[1, wid, pl.ds(w * W, W)],
                              ibuf.at[w % 2], semi).start()

    idx_start(0)

    @pl.loop(0, C // L + 1)
    def _fill(i):
        ones_v[pl.ds(i * L, L)] = jnp.ones((L,), jnp.float32)

    @pl.loop(0, RPS // L)
    def _fillz(i):
        zeros_v[pl.ds(i * L, L)] = jnp.zeros((L,), jnp.float32)

    pltpu.sync_copy(zeros_v, acc.at[pl.ds(s * RPS, RPS)])
    plsc.subcore_barrier()

    @pl.loop(0, NWIN)
    def _win(w):
        pltpu.make_async_copy(edge_hbm.at[1, wid, pl.ds(0, W)],
                              ibuf.at[0], semi).wait()

        @pl.when(w + 1 < NWIN)
        def _():
            idx_start(w + 1)

        wb = w % 2
        for j in range(W):
            pltpu.async_copy(ones_v.at[pl.ds(0, C)], acc.at[ibuf.at[wb, j]],
                             sems, add=True)

    @pl.loop(0, CHUNKS)
    def _drain(j):
        pltpu.make_async_copy(ones_v.at[pl.ds(0, C)], acc.at[ibuf.at[0, 0]],
                              sems).wait()

    plsc.subcore_barrier()
    pltpu.sync_copy(acc.at[pl.ds(s * RPS, RPS)], out_hbm.at[c, pl.ds(s * RPS, RPS)])


@jax.jit
def _deg_call(edges):
    return pl.kernel(
        _deg_body,
        out_type=jax.ShapeDtypeStruct((NC, N_PAD), jnp.float32),
        mesh=_mesh(),
        scratch_types=[
            pltpu.VMEM((2, W, C), jnp.int32),
            pltpu.VMEM((C // L * L + L,), jnp.float32),
            pltpu.VMEM((RPS,), jnp.float32),
            pltpu.VMEM_SHARED((N_PAD,), jnp.float32),
            pltpu.SemaphoreType.DMA,
            pltpu.SemaphoreType.DMA,
        ],
    )(edges)


def _agg_body(table_hbm, edge_hbm, out_hbm, ibuf, rows, zbuf, acc,
              semi, sem0, sem1, semS0, semS1):
    c = lax.axis_index("c")
    s = lax.axis_index("s")
    wid = s * NC + c

    def idx_start(w):
        pltpu.make_async_copy(edge_hbm.at[0, wid, pl.ds(w * W, W)],
                              ibuf.at[w % 2, 0], semi).start()
        pltpu.make_async_copy(edge_hbm.at[1, wid, pl.ds(w * W, W)],
                              ibuf.at[w % 2, 1], semi).start()

    def idx_wait():
        pltpu.make_async_copy(edge_hbm.at[0, 0, pl.ds(0, W)],
                              ibuf.at[0, 0], semi).wait()
        pltpu.make_async_copy(edge_hbm.at[0, 0, pl.ds(0, W)],
                              ibuf.at[0, 1], semi).wait()

    def g_start(t, b, sem):
        pltpu.make_async_copy(table_hbm.at[ibuf.at[(t // W) % 2, 0, t % W]],
                              rows.at[b], sem).start()

    def g_wait(b, sem):
        pltpu.make_async_copy(table_hbm.at[ibuf.at[0, 0, 0]],
                              rows.at[b], sem).wait()

    def scat_start(t, b, sem):
        pltpu.async_copy(rows.at[b], acc.at[ibuf.at[(t // W) % 2, 1, t % W]],
                         sem, add=True)

    def scat_wait(b, sem):
        pltpu.make_async_copy(rows.at[b], acc.at[ibuf.at[0, 1, 0]],
                              sem).wait()

    idx_start(0)

    @pl.loop(0, SUB)
    def _fill(i):
        for k in range(D // L):
            zbuf[i, pl.ds(k * L, L)] = jnp.zeros((L,), jnp.float32)

    idx_wait()
    idx_start(1)
    g_start(0, 0, sem0)
    g_start(1, 1, sem1)

    @pl.loop(0, RPS // SUB)
    def _zero(k):
        pltpu.sync_copy(zbuf, acc.at[pl.ds(s * RPS + k * SUB, SUB)])

    plsc.subcore_barrier()

    @pl.loop(0, CHUNKS, step=2)
    def _main(t):
        g_wait(0, sem0)
        scat_start(t, 0, semS0)
        bnd = ((t + 2) % W == 0) & (t + 2 < CHUNKS)

        @pl.when(bnd)
        def _():
            idx_wait()

        g_wait(1, sem1)
        scat_start(t + 1, 1, semS1)

        scat_wait(0, semS0)

        @pl.when(t + 2 < CHUNKS)
        def _():
            g_start(t + 2, 0, sem0)

        scat_wait(1, semS1)

        @pl.when(t + 3 < CHUNKS)
        def _():
            g_start(t + 3, 1, sem1)

        @pl.when(bnd & ((t + 2) // W + 1 < NWIN))
        def _():
            idx_start((t + 2) // W + 1)

    plsc.subcore_barrier()
    pltpu.sync_copy(acc.at[pl.ds(s * RPS, RPS)],
                    out_hbm.at[c, pl.ds(s * RPS, RPS)])


@jax.jit
def _agg_call(table, edges):
    return pl.kernel(
        _agg_body,
        out_type=jax.ShapeDtypeStruct((NC, N_PAD, D), jnp.float32),
        mesh=_mesh(),
        scratch_types=[
            pltpu.VMEM((2, 2, W, C), jnp.int32),
            pltpu.VMEM((2, C, D), jnp.float32),
            pltpu.VMEM((SUB, D), jnp.float32),
            pltpu.VMEM_SHARED((N_PAD, D), jnp.float32),
            pltpu.SemaphoreType.DMA,
            pltpu.SemaphoreType.DMA,
            pltpu.SemaphoreType.DMA,
            pltpu.SemaphoreType.DMA,
            pltpu.SemaphoreType.DMA,
        ],
    )(table, edges)


def _dinv_col(degp_ref):
    deg = degp_ref[:, 0:1] + degp_ref[:, 1:2] + 1.0
    return lax.rsqrt(deg)


def _tc1_body(degp_ref, x_ref, xs_ref):
    xs_ref[...] = x_ref[...] * _dinv_col(degp_ref)


@jax.jit
def _tc1_call(degp, x):
    return pl.pallas_call(
        _tc1_body,
        grid=(N // R_BLK,),
        in_specs=[
            pl.BlockSpec((R_BLK, NC), lambda i: (i, 0)),
            pl.BlockSpec((R_BLK, D), lambda i: (i, 0)),
        ],
        out_specs=pl.BlockSpec((R_BLK, D), lambda i: (i, 0)),
        out_shape=jax.ShapeDtypeStruct((N, D), jnp.float32),
    )(degp, x)


def _tc2_body(degp_ref, p_ref, xs_ref, w_ref, b_ref, hs_ref):
    dinv = _dinv_col(degp_ref)
    t = dinv * (p_ref[0] + p_ref[1] + xs_ref[...])
    h = jnp.dot(t, w_ref[...], preferred_element_type=jnp.float32) + b_ref[...]
    hs_ref[...] = dinv * jnp.maximum(h, 0.0)


@jax.jit
def _tc2_call(degp, p, xs, W1, b1):
    return pl.pallas_call(
        _tc2_body,
        grid=(N // R_BLK,),
        in_specs=[
            pl.BlockSpec((R_BLK, NC), lambda i: (i, 0)),
            pl.BlockSpec((NC, R_BLK, D), lambda i: (0, i, 0)),
            pl.BlockSpec((R_BLK, D), lambda i: (i, 0)),
            pl.BlockSpec((D, D), lambda i: (0, 0)),
            pl.BlockSpec((1, D), lambda i: (0, 0)),
        ],
        out_specs=pl.BlockSpec((R_BLK, D), lambda i: (i, 0)),
        out_shape=jax.ShapeDtypeStruct((N, D), jnp.float32),
    )(degp, p, xs, W1, b1)


def _tc3_body(degp_ref, q_ref, hs_ref, wmu_ref, wls_ref, bmu_ref, bls_ref,
              mu_ref, ls_ref):
    g = _dinv_col(degp_ref) * (q_ref[0] + q_ref[1] + hs_ref[...])
    mu_ref[...] = (
        jnp.dot(g, wmu_ref[...], preferred_element_type=jnp.float32)
        + bmu_ref[...]
    )
    ls_ref[...] = (
        jnp.dot(g, wls_ref[...], preferred_element_type=jnp.float32)
        + bls_ref[...]
    )


@jax.jit
def _tc3_call(degp, q, hs, W_mu, W_ls, b_mu, b_ls):
    return pl.pallas_call(
        _tc3_body,
        grid=(N // R_BLK,),
        in_specs=[
            pl.BlockSpec((R_BLK, NC), lambda i: (i, 0)),
            pl.BlockSpec((NC, R_BLK, D), lambda i: (0, i, 0)),
            pl.BlockSpec((R_BLK, D), lambda i: (i, 0)),
            pl.BlockSpec((D, D_OUT), lambda i: (0, 0)),
            pl.BlockSpec((D, D_OUT), lambda i: (0, 0)),
            pl.BlockSpec((1, D_OUT), lambda i: (0, 0)),
            pl.BlockSpec((1, D_OUT), lambda i: (0, 0)),
        ],
        out_specs=[
            pl.BlockSpec((R_BLK, D_OUT), lambda i: (i, 0)),
            pl.BlockSpec((R_BLK, D_OUT), lambda i: (i, 0)),
        ],
        out_shape=[
            jax.ShapeDtypeStruct((N, D_OUT), jnp.float32),
            jax.ShapeDtypeStruct((N, D_OUT), jnp.float32),
        ],
    )(degp, q, hs, W_mu, W_ls, b_mu, b_ls)


def kernel(x, edge_index, W1, b1, W_mu, b_mu, W_ls, b_ls):
    edges = edge_index.reshape(2, NW, CHUNKS, C)
    degp = _deg_call(edges).T
    xs = _tc1_call(degp, x)
    p = _agg_call(xs, edges)
    hs = _tc2_call(degp, p, xs, W1, b1.reshape(1, D))
    q = _agg_call(hs, edges)
    return _tc3_call(degp, q, hs, W_mu, W_ls,
                     b_mu.reshape(1, D_OUT), b_ls.reshape(1, D_OUT))


# speedup vs baseline: 1.2588x; 1.2588x over previous
import jax
import jax.numpy as jnp
from jax import lax
from jax.experimental import pallas as pl
from jax.experimental.pallas import tpu as pltpu
from jax.experimental.pallas import tpu_sc as plsc

N = 10000
E = 320000
D = 128
D_OUT = 64

NC = 2
NS = 16
L = 16
NW = NC * NS

C = 125
W = 8
NWIN = 10
CHUNKS = W * NWIN
N_PAD = 10240
RPS = N_PAD // NS
R_BLK = 2000
SUB = 64


def _mesh():
    return plsc.VectorSubcoreMesh(core_axis_name="c", subcore_axis_name="s")


def _deg_body(edge_hbm, out_hbm, ibuf, ones_v, zeros_v, acc, semi, sems):
    c = lax.axis_index("c")
    s = lax.axis_index("s")
    wid = s * NC + c

    def idx_start(w):
        pltpu.make_async_copy(edge_hbm.at[1, wid, pl.ds(w * W, W)],
                              ibuf.at[w % 2], semi).start()

    idx_start(0)

    @pl.loop(0, C // L + 1)
    def _fill(i):
        ones_v[pl.ds(i * L, L)] = jnp.ones((L,), jnp.float32)

    @pl.loop(0, RPS // L)
    def _fillz(i):
        zeros_v[pl.ds(i * L, L)] = jnp.zeros((L,), jnp.float32)

    pltpu.sync_copy(zeros_v, acc.at[pl.ds(s * RPS, RPS)])
    plsc.subcore_barrier()

    @pl.loop(0, NWIN)
    def _win(w):
        pltpu.make_async_copy(edge_hbm.at[1, wid, pl.ds(0, W)],
                              ibuf.at[0], semi).wait()

        @pl.when(w + 1 < NWIN)
        def _():
            idx_start(w + 1)

        wb = w % 2
        for j in range(W):
            pltpu.async_copy(ones_v.at[pl.ds(0, C)], acc.at[ibuf.at[wb, j]],
                             sems, add=True)

    @pl.loop(0, CHUNKS)
    def _drain(j):
        pltpu.make_async_copy(ones_v.at[pl.ds(0, C)], acc.at[ibuf.at[0, 0]],
                              sems).wait()

    plsc.subcore_barrier()
    pltpu.sync_copy(acc.at[pl.ds(s * RPS, RPS)], out_hbm.at[c, pl.ds(s * RPS, RPS)])


@jax.jit
def _deg_call(edges):
    return pl.kernel(
        _deg_body,
        out_type=jax.ShapeDtypeStruct((NC, N_PAD), jnp.float32),
        mesh=_mesh(),
        scratch_types=[
            pltpu.VMEM((2, W, C), jnp.int32),
            pltpu.VMEM((C // L * L + L,), jnp.float32),
            pltpu.VMEM((RPS,), jnp.float32),
            pltpu.VMEM_SHARED((N_PAD,), jnp.float32),
            pltpu.SemaphoreType.DMA,
            pltpu.SemaphoreType.DMA,
        ],
    )(edges)


def _agg_body(table_hbm, edge_hbm, out_hbm, ibuf, rows, zbuf, acc,
              semi, sem0, sem1):
    c = lax.axis_index("c")
    s = lax.axis_index("s")
    wid = s * NC + c

    def idx_start(w):
        pltpu.make_async_copy(edge_hbm.at[0, wid, pl.ds(w * W, W)],
                              ibuf.at[w % 2, 0], semi).start()
        pltpu.make_async_copy(edge_hbm.at[1, wid, pl.ds(w * W, W)],
                              ibuf.at[w % 2, 1], semi).start()

    def idx_wait():
        pltpu.make_async_copy(edge_hbm.at[0, 0, pl.ds(0, W)],
                              ibuf.at[0, 0], semi).wait()
        pltpu.make_async_copy(edge_hbm.at[0, 0, pl.ds(0, W)],
                              ibuf.at[0, 1], semi).wait()

    def g_start(t, b, sem):
        pltpu.make_async_copy(table_hbm.at[ibuf.at[(t // W) % 2, 0, t % W]],
                              rows.at[b], sem).start()

    def g_wait(b, sem):
        pltpu.make_async_copy(table_hbm.at[ibuf.at[0, 0, 0]],
                              rows.at[b], sem).wait()

    def scat(t, b):
        pltpu.sync_copy(rows.at[b], acc.at[ibuf.at[(t // W) % 2, 1, t % W]],
                        add=True)

    idx_start(0)

    @pl.loop(0, SUB)
    def _fill(i):
        for k in range(D // L):
            zbuf[i, pl.ds(k * L, L)] = jnp.zeros((L,), jnp.float32)

    idx_wait()
    idx_start(1)
    g_start(0, 0, sem0)
    g_start(1, 1, sem1)

    @pl.loop(0, RPS // SUB)
    def _zero(k):
        pltpu.sync_copy(zbuf, acc.at[pl.ds(s * RPS + k * SUB, SUB)])

    plsc.subcore_barrier()

    @pl.loop(0, CHUNKS, step=2)
    def _main(t):
        g_wait(0, sem0)
        scat(t, 0)
        bnd = ((t + 2) % W == 0) & (t + 2 < CHUNKS)

        @pl.when(bnd)
        def _():
            idx_wait()

        @pl.when(t + 2 < CHUNKS)
        def _():
            g_start(t + 2, 0, sem0)

        g_wait(1, sem1)
        scat(t + 1, 1)

        @pl.when(t + 3 < CHUNKS)
        def _():
            g_start(t + 3, 1, sem1)

        @pl.when(bnd & ((t + 2) // W + 1 < NWIN))
        def _():
            idx_start((t + 2) // W + 1)

    plsc.subcore_barrier()
    pltpu.sync_copy(acc.at[pl.ds(s * RPS, RPS)],
                    out_hbm.at[c, pl.ds(s * RPS, RPS)])


@jax.jit
def _agg_call(table, edges):
    return pl.kernel(
        _agg_body,
        out_type=jax.ShapeDtypeStruct((NC, N_PAD, D), jnp.float32),
        mesh=_mesh(),
        scratch_types=[
            pltpu.VMEM((2, 2, W, C), jnp.int32),
            pltpu.VMEM((2, C, D), jnp.float32),
            pltpu.VMEM((SUB, D), jnp.float32),
            pltpu.VMEM_SHARED((N_PAD, D), jnp.float32),
            pltpu.SemaphoreType.DMA,
            pltpu.SemaphoreType.DMA,
            pltpu.SemaphoreType.DMA,
        ],
    )(table, edges)


def _dinv_col(degp_ref):
    deg = degp_ref[:, 0:1] + degp_ref[:, 1:2] + 1.0
    return lax.rsqrt(deg)


def _tc1_body(degp_ref, x_ref, xs_ref):
    xs_ref[...] = x_ref[...] * _dinv_col(degp_ref)


@jax.jit
def _tc1_call(degp, x):
    return pl.pallas_call(
        _tc1_body,
        grid=(N // R_BLK,),
        in_specs=[
            pl.BlockSpec((R_BLK, NC), lambda i: (i, 0)),
            pl.BlockSpec((R_BLK, D), lambda i: (i, 0)),
        ],
        out_specs=pl.BlockSpec((R_BLK, D), lambda i: (i, 0)),
        out_shape=jax.ShapeDtypeStruct((N, D), jnp.float32),
    )(degp, x)


def _tc2_body(degp_ref, p_ref, xs_ref, w_ref, b_ref, hs_ref):
    dinv = _dinv_col(degp_ref)
    t = dinv * (p_ref[0] + p_ref[1] + xs_ref[...])
    h = jnp.dot(t, w_ref[...], preferred_element_type=jnp.float32) + b_ref[...]
    hs_ref[...] = dinv * jnp.maximum(h, 0.0)


@jax.jit
def _tc2_call(degp, p, xs, W1, b1):
    return pl.pallas_call(
        _tc2_body,
        grid=(N // R_BLK,),
        in_specs=[
            pl.BlockSpec((R_BLK, NC), lambda i: (i, 0)),
            pl.BlockSpec((NC, R_BLK, D), lambda i: (0, i, 0)),
            pl.BlockSpec((R_BLK, D), lambda i: (i, 0)),
            pl.BlockSpec((D, D), lambda i: (0, 0)),
            pl.BlockSpec((1, D), lambda i: (0, 0)),
        ],
        out_specs=pl.BlockSpec((R_BLK, D), lambda i: (i, 0)),
        out_shape=jax.ShapeDtypeStruct((N, D), jnp.float32),
    )(degp, p, xs, W1, b1)


def _tc3_body(degp_ref, q_ref, hs_ref, wmu_ref, wls_ref, bmu_ref, bls_ref,
              mu_ref, ls_ref):
    g = _dinv_col(degp_ref) * (q_ref[0] + q_ref[1] + hs_ref[...])
    mu_ref[...] = (
        jnp.dot(g, wmu_ref[...], preferred_element_type=jnp.float32)
        + bmu_ref[...]
    )
    ls_ref[...] = (
        jnp.dot(g, wls_ref[...], preferred_element_type=jnp.float32)
        + bls_ref[...]
    )


@jax.jit
def _tc3_call(degp, q, hs, W_mu, W_ls, b_mu, b_ls):
    return pl.pallas_call(
        _tc3_body,
        grid=(N // R_BLK,),
        in_specs=[
            pl.BlockSpec((R_BLK, NC), lambda i: (i, 0)),
            pl.BlockSpec((NC, R_BLK, D), lambda i: (0, i, 0)),
            pl.BlockSpec((R_BLK, D), lambda i: (i, 0)),
            pl.BlockSpec((D, D_OUT), lambda i: (0, 0)),
            pl.BlockSpec((D, D_OUT), lambda i: (0, 0)),
            pl.BlockSpec((1, D_OUT), lambda i: (0, 0)),
            pl.BlockSpec((1, D_OUT), lambda i: (0, 0)),
        ],
        out_specs=[
            pl.BlockSpec((R_BLK, D_OUT), lambda i: (i, 0)),
            pl.BlockSpec((R_BLK, D_OUT), lambda i: (i, 0)),
        ],
        out_shape=[
            jax.ShapeDtypeStruct((N, D_OUT), jnp.float32),
            jax.ShapeDtypeStruct((N, D_OUT), jnp.float32),
        ],
    )(degp, q, hs, W_mu, W_ls, b_mu, b_ls)


def kernel(x, edge_index, W1, b1, W_mu, b_mu, W_ls, b_ls):
    edges = edge_index.reshape(2, NW, CHUNKS, C)
    degp = _deg_call(edges).T
    xs = _tc1_call(degp, x)
    p = _agg_call(xs, edges)
    hs = _tc2_call(degp, p, xs, W1, b1.reshape(1, D))
    q = _agg_call(hs, edges)
    return _tc3_call(degp, q, hs, W_mu, W_ls,
                     b_mu.reshape(1, D_OUT), b_ls.reshape(1, D_OUT))
